# Initial kernel scaffold; baseline (speedup 1.0000x reference)
import jax
import jax.numpy as jnp
from jax.experimental import pallas as pl


def kernel(x, w_ih0, w_hh0, w_fused, b, ln_g, ln_b, w_out, b_out):
    raise NotImplementedError("write your pallas kernel here")


# speedup vs baseline: 3.2211x; 3.2211x over previous
import jax
import jax.numpy as jnp
from jax.experimental import pallas as pl
from jax.experimental.pallas import tpu as pltpu

IN_FEATS = 6
F_PAD = 8
HIDDEN = 64
NUM_LAYERS = 4
LABEL = 20
LN_EPS = 1e-5
BT = 256


def _lstm_wavefront_kernel(x_ref, w0t_ref, wft_ref, bt_ref, ln_g_ref,
                           ln_b_ref, w_out_ref, b_out_ref, out_ref):
    T = x_ref.shape[0]
    Bt = x_ref.shape[2]
    L, H = NUM_LAYERS, HIDDEN
    f32 = jnp.float32
    bf16 = jnp.bfloat16

    row = jax.lax.broadcasted_iota(jnp.int32, (2 * H, 1), 0)
    hi = row >= H
    pre_scale = jnp.where(hi, 2.0, 1.0).astype(f32)
    post_shift = jnp.where(hi, 1.0, 0.0).astype(f32)

    w0t = w0t_ref[...]
    wft = [wft_ref[l] for l in range(L - 1)]
    bias = [bt_ref[l] for l in range(L)]

    h_bf = [jnp.zeros((H, Bt), bf16) for _ in range(L)]
    c = [jnp.zeros((H, Bt), f32) for _ in range(L)]
    below = [[None] * T for _ in range(L - 1)]
    h_top = [jnp.zeros((H, Bt), f32)]

    def cell(l, t):
        if l == 0:
            hin = jnp.concatenate([h_bf[0], x_ref[t]], axis=0)
            gates = jnp.dot(w0t, hin, preferred_element_type=f32) + bias[0]
        else:
            hin = jnp.concatenate([below[l - 1][t], h_bf[l]], axis=0)
            gates = jnp.dot(wft[l - 1], hin, preferred_element_type=f32) + bias[l]

        v0 = gates[:2 * H]
        v1 = gates[2 * H:]
        s0 = jax.nn.sigmoid(v0)
        s1 = jax.nn.sigmoid(v1 * pre_scale) * pre_scale - post_shift
        p = s0 * s1
        c_new = s0[:H] * c[l] + p[H:]
        h_new = s1[:H] * jnp.tanh(c_new)
        c[l] = c_new
        hb = h_new.astype(bf16)
        h_bf[l] = hb
        if l < L - 1:
            below[l][t] = hb
        else:
            h_top[0] = h_new

    for d in range(L + T - 1):
        for l in range(L):
            t = d - l
            if 0 <= t < T:
                cell(l, t)

    h_last = h_top[0]
    mu = jnp.mean(h_last, axis=0, keepdims=True)
    var = jnp.mean((h_last - mu) ** 2, axis=0, keepdims=True)
    xn = (h_last - mu) * jax.lax.rsqrt(var + LN_EPS)
    xn = xn * ln_g_ref[...] + ln_b_ref[...]

    out = jnp.dot(w_out_ref[...], xn, preferred_element_type=f32) + b_out_ref[...]
    out_ref[...] = out.astype(out_ref.dtype)


@jax.jit
def kernel(x, w_ih0, w_hh0, w_fused, b, ln_g, ln_b, w_out, b_out):
    B, T, F = x.shape
    H, L = HIDDEN, NUM_LAYERS
    G = 4 * H
    bt = BT if B >= BT else max(8, (B + 7) // 8 * 8)
    nb = pl.cdiv(B, bt)
    b_pad = bt * nb

    x_p = jnp.zeros((T, F_PAD, b_pad), jnp.bfloat16)
    x_p = x_p.at[:, :F, :B].set(jnp.transpose(x.astype(jnp.bfloat16), (1, 2, 0)))

    w0t = jnp.concatenate([w_hh0, w_ih0], axis=0).T
    wft = jnp.transpose(w_fused, (0, 2, 1))
    b_t = jnp.transpose(b, (0, 2, 1))

    flops = (2 * b_pad * T * (F_PAD + H) * G
             + 2 * b_pad * T * (L - 1) * 2 * H * G
             + 2 * b_pad * H * LABEL)
    transcendentals = b_pad * T * L * 5 * H
    bytes_accessed = int(x_p.size * 2 + w_fused.size * 2 + b_pad * LABEL * 4)

    def resident(a):
        nd = a.ndim
        return pl.BlockSpec(a.shape, lambda i, nd=nd: (0,) * nd)

    out = pl.pallas_call(
        _lstm_wavefront_kernel,
        out_shape=jax.ShapeDtypeStruct((LABEL, b_pad), jnp.float32),
        grid=(nb,),
        in_specs=[
            pl.BlockSpec((T, F_PAD, bt), lambda i: (0, 0, i)),
            resident(w0t),
            resident(wft),
            resident(b_t),
            resident(ln_g.T),
            resident(ln_b.T),
            resident(w_out.T),
            resident(b_out.T),
        ],
        out_specs=pl.BlockSpec((LABEL, bt), lambda i: (0, i)),
        cost_estimate=pl.CostEstimate(flops=flops,
                                      transcendentals=transcendentals,
                                      bytes_accessed=bytes_accessed),
        compiler_params=pltpu.CompilerParams(
            dimension_semantics=("parallel",)),
    )(x_p, w0t, wft, b_t, ln_g.T, ln_b.T, w_out.T, b_out.T)
    return out[:, :B].T


# speedup vs baseline: 5.0470x; 1.5668x over previous
import jax
import jax.numpy as jnp
from jax.experimental import pallas as pl
from jax.experimental.pallas import tpu as pltpu

IN_FEATS = 6
F_PAD = 8
HIDDEN = 64
NUM_LAYERS = 4
LABEL = 20
LN_EPS = 1e-5
BT = 512


def _lstm_wavefront_kernel(x_ref, w0t_ref, wft_ref, bt_ref, ln_g_ref,
                           ln_b_ref, w_out_ref, b_out_ref, out_ref):
    T = x_ref.shape[0]
    Bt = x_ref.shape[2]
    L, H = NUM_LAYERS, HIDDEN
    f32 = jnp.float32
    bf16 = jnp.bfloat16

    w0t = w0t_ref[...]
    wft = [wft_ref[l] for l in range(L - 1)]
    bias = [bt_ref[l] for l in range(L)]

    h_bf = [jnp.zeros((H, Bt), bf16) for _ in range(L)]
    c = [jnp.zeros((H, Bt), f32) for _ in range(L)]
    below = [[None] * T for _ in range(L - 1)]
    h_top = [jnp.zeros((H, Bt), f32)]

    def cell(l, t):
        if l == 0:
            hin = jnp.concatenate([h_bf[0], x_ref[t]], axis=0)
            gates = jnp.dot(w0t, hin, preferred_element_type=f32) + bias[0]
        else:
            hin = jnp.concatenate([below[l - 1][t], h_bf[l]], axis=0)
            gates = jnp.dot(wft[l - 1], hin, preferred_element_type=f32) + bias[l]

        u = jnp.tanh(gates)
        c_new = 0.5 * (c[l] * (1.0 + u[:H]) + u[3 * H:] * (1.0 + u[H:2 * H]))
        tc = jnp.tanh(c_new)
        h2 = u[2 * H:3 * H] * tc + tc
        c[l] = c_new
        hb = h2.astype(bf16)
        h_bf[l] = hb
        if l < L - 1:
            below[l][t] = hb
        else:
            h_top[0] = h2

    for d in range(L + T - 1):
        for l in range(L):
            t = d - l
            if 0 <= t < T:
                cell(l, t)

    h_last = h_top[0]
    mu = jnp.mean(h_last, axis=0, keepdims=True)
    var = jnp.mean((h_last - mu) ** 2, axis=0, keepdims=True)
    xn = (h_last - mu) * jax.lax.rsqrt(var + LN_EPS * 4.0)
    xn = xn * ln_g_ref[...] + ln_b_ref[...]

    out = jnp.dot(w_out_ref[...], xn, preferred_element_type=f32) + b_out_ref[...]
    out_ref[...] = out.astype(out_ref.dtype)


@jax.jit
def kernel(x, w_ih0, w_hh0, w_fused, b, ln_g, ln_b, w_out, b_out):
    B, T, F = x.shape
    H, L = HIDDEN, NUM_LAYERS
    G = 4 * H
    bt = BT if B >= BT else max(8, (B + 7) // 8 * 8)
    nb = pl.cdiv(B, bt)
    b_pad = bt * nb

    x_p = jnp.zeros((T, F_PAD, b_pad), jnp.bfloat16)
    x_p = x_p.at[:, :F, :B].set(jnp.transpose(x.astype(jnp.bfloat16), (1, 2, 0)))

    gate_scale = jnp.concatenate(
        [jnp.full((3 * H, 1), 0.5, jnp.float32),
         jnp.ones((H, 1), jnp.float32)], axis=0)
    h_in_scale = jnp.concatenate(
        [jnp.full((1, H), 0.5, jnp.float32),
         jnp.ones((1, F_PAD), jnp.float32)], axis=1)
    w0t = (jnp.concatenate([w_hh0, w_ih0], axis=0).T
           * (gate_scale * h_in_scale).astype(jnp.bfloat16))
    wft = (jnp.transpose(w_fused, (0, 2, 1))
           * (gate_scale * 0.5).astype(jnp.bfloat16))
    b_t = jnp.transpose(b, (0, 2, 1)) * gate_scale

    flops = (2 * b_pad * T * (F_PAD + H) * G
             + 2 * b_pad * T * (L - 1) * 2 * H * G
             + 2 * b_pad * H * LABEL)
    transcendentals = b_pad * T * L * 5 * H
    bytes_accessed = int(x_p.size * 2 + w_fused.size * 2 + b_pad * LABEL * 4)

    def resident(a):
        nd = a.ndim
        return pl.BlockSpec(a.shape, lambda i, nd=nd: (0,) * nd)

    out = pl.pallas_call(
        _lstm_wavefront_kernel,
        out_shape=jax.ShapeDtypeStruct((LABEL, b_pad), jnp.float32),
        grid=(nb,),
        in_specs=[
            pl.BlockSpec((T, F_PAD, bt), lambda i: (0, 0, i)),
            resident(w0t),
            resident(wft),
            resident(b_t),
            resident(ln_g.T),
            resident(ln_b.T),
            resident(w_out.T),
            resident(b_out.T),
        ],
        out_specs=pl.BlockSpec((LABEL, bt), lambda i: (0, i)),
        cost_estimate=pl.CostEstimate(flops=flops,
                                      transcendentals=transcendentals,
                                      bytes_accessed=bytes_accessed),
        compiler_params=pltpu.CompilerParams(
            dimension_semantics=("parallel",)),
    )(x_p, w0t, wft, b_t, ln_g.T, ln_b.T, w_out.T, b_out.T)
    return out[:, :B].T


# speedup vs baseline: 5.6057x; 1.1107x over previous
import jax
import jax.numpy as jnp
from jax.experimental import pallas as pl
from jax.experimental.pallas import tpu as pltpu

IN_FEATS = 6
F_PAD = 8
HIDDEN = 64
NUM_LAYERS = 4
LABEL = 20
LN_EPS = 1e-5
BT = 512


def _lstm_wavefront_kernel(x_ref, w0t_ref, wft_ref, bt_ref, ln_g_ref,
                           ln_b_ref, w_out_ref, b_out_ref, out_ref):
    T = x_ref.shape[0]
    Bt = x_ref.shape[2]
    L, H = NUM_LAYERS, HIDDEN
    f32 = jnp.float32
    bf16 = jnp.bfloat16

    w0t = w0t_ref[...]
    wft = [wft_ref[l] for l in range(L - 1)]
    bias = [None] + [bt_ref[l] for l in range(1, L)]

    h_bf = [jnp.zeros((H, Bt), bf16) for _ in range(L)]
    c = [jnp.zeros((H, Bt), f32) for _ in range(L)]
    below = [[None] * T for _ in range(L - 1)]
    h_top = [jnp.zeros((H, Bt), f32)]

    one = jnp.bfloat16(1.0)
    half = jnp.bfloat16(0.5)

    def cell(l, t):
        if l == 0:
            hin = jnp.concatenate([h_bf[0], x_ref[t]], axis=0)
            gb = jnp.dot(w0t, hin, preferred_element_type=f32).astype(bf16)
        else:
            hin = jnp.concatenate([below[l - 1][t], h_bf[l]], axis=0)
            gb = (jnp.dot(wft[l - 1], hin, preferred_element_type=f32)
                  .astype(bf16) + bias[l])

        u = jnp.tanh(gb)
        tfh = (one + u[:H]) * half
        pg = u[3 * H:] * (one + u[H:2 * H])
        c2_new = c[l] * tfh.astype(f32) + pg.astype(f32)
        tc = jnp.tanh(c2_new.astype(bf16) * half)
        h2 = u[2 * H:3 * H] * tc + tc
        c[l] = c2_new
        h_bf[l] = h2
        if l < L - 1:
            below[l][t] = h2
        else:
            h_top[0] = h2.astype(f32)

    for d in range(L + T - 1):
        for l in range(L):
            t = d - l
            if 0 <= t < T:
                cell(l, t)

    h_last = h_top[0]
    mu = jnp.mean(h_last, axis=0, keepdims=True)
    var = jnp.mean((h_last - mu) ** 2, axis=0, keepdims=True)
    xn = (h_last - mu) * jax.lax.rsqrt(var + LN_EPS * 4.0)
    xn = xn * ln_g_ref[...] + ln_b_ref[...]

    out = jnp.dot(w_out_ref[...], xn, preferred_element_type=f32) + b_out_ref[...]
    out_ref[...] = out.astype(out_ref.dtype)


@jax.jit
def kernel(x, w_ih0, w_hh0, w_fused, b, ln_g, ln_b, w_out, b_out):
    B, T, F = x.shape
    H, L = HIDDEN, NUM_LAYERS
    G = 4 * H
    bt = BT if B >= BT else max(8, (B + 7) // 8 * 8)
    nb = pl.cdiv(B, bt)
    b_pad = bt * nb

    x_p = jnp.zeros((T, F_PAD, b_pad), jnp.bfloat16)
    x_p = x_p.at[:, :F, :B].set(jnp.transpose(x.astype(jnp.bfloat16), (1, 2, 0)))
    x_p = x_p.at[:, F, :].set(jnp.bfloat16(1.0))

    gate_scale = jnp.concatenate(
        [jnp.full((3 * H, 1), 0.5, jnp.float32),
         jnp.ones((H, 1), jnp.float32)], axis=0)
    h_in_scale = jnp.concatenate(
        [jnp.full((1, H), 0.5, jnp.float32),
         jnp.ones((1, F_PAD), jnp.float32)], axis=1)
    w0_rows = jnp.concatenate(
        [w_hh0.astype(jnp.float32),
         w_ih0[:F].astype(jnp.float32),
         b[0],
         jnp.zeros((1, G), jnp.float32)], axis=0)
    w0t = (w0_rows.T * gate_scale * h_in_scale).astype(jnp.bfloat16)
    wft = (jnp.transpose(w_fused, (0, 2, 1))
           * (gate_scale * 0.5).astype(jnp.bfloat16))
    b_t = (jnp.transpose(b, (0, 2, 1)) * gate_scale).astype(jnp.bfloat16)

    flops = (2 * b_pad * T * (F_PAD + H) * G
             + 2 * b_pad * T * (L - 1) * 2 * H * G
             + 2 * b_pad * H * LABEL)
    transcendentals = b_pad * T * L * 5 * H
    bytes_accessed = int(x_p.size * 2 + w_fused.size * 2 + b_pad * LABEL * 4)

    def resident(a):
        nd = a.ndim
        return pl.BlockSpec(a.shape, lambda i, nd=nd: (0,) * nd)

    out = pl.pallas_call(
        _lstm_wavefront_kernel,
        out_shape=jax.ShapeDtypeStruct((LABEL, b_pad), jnp.float32),
        grid=(nb,),
        in_specs=[
            pl.BlockSpec((T, F_PAD, bt), lambda i: (0, 0, i)),
            resident(w0t),
            resident(wft),
            resident(b_t),
            resident(ln_g.T),
            resident(ln_b.T),
            resident(w_out.T),
            resident(b_out.T),
        ],
        out_specs=pl.BlockSpec((LABEL, bt), lambda i: (0, i)),
        cost_estimate=pl.CostEstimate(flops=flops,
                                      transcendentals=transcendentals,
                                      bytes_accessed=bytes_accessed),
        compiler_params=pltpu.CompilerParams(
            dimension_semantics=("parallel",)),
    )(x_p, w0t, wft, b_t, ln_g.T, ln_b.T, w_out.T, b_out.T)
    return out[:, :B].T


# speedup vs baseline: 5.7998x; 1.0346x over previous
import jax
import jax.numpy as jnp
from jax.experimental import pallas as pl
from jax.experimental.pallas import tpu as pltpu

IN_FEATS = 6
F_PAD = 8
HIDDEN = 64
NUM_LAYERS = 4
LABEL = 20
LN_EPS = 1e-5
BT = 1024


def _lstm_wavefront_kernel(x_ref, w0t_ref, wft_ref, bt_ref, ln_g_ref,
                           ln_b_ref, w_out_ref, b_out_ref, out_ref):
    T = x_ref.shape[0]
    Bt = x_ref.shape[2]
    L, H = NUM_LAYERS, HIDDEN
    f32 = jnp.float32
    bf16 = jnp.bfloat16

    w0t = w0t_ref[...]
    wft = [wft_ref[l] for l in range(L - 1)]
    bias = [None] + [bt_ref[l] for l in range(1, L)]

    h_bf = [jnp.zeros((H, Bt), bf16) for _ in range(L)]
    c = [jnp.zeros((H, Bt), f32) for _ in range(L)]
    below = [[None] * T for _ in range(L - 1)]
    h_top = [jnp.zeros((H, Bt), f32)]

    one = jnp.bfloat16(1.0)
    half = jnp.bfloat16(0.5)

    def cell(l, t):
        if l == 0:
            hin = jnp.concatenate([h_bf[0], x_ref[t]], axis=0)
            gb = jnp.dot(w0t, hin, preferred_element_type=f32).astype(bf16)
        else:
            hin = jnp.concatenate([below[l - 1][t], h_bf[l]], axis=0)
            gb = (jnp.dot(wft[l - 1], hin, preferred_element_type=f32)
                  .astype(bf16) + bias[l])

        u = jnp.tanh(gb)
        tfh = (one + u[:H]) * half
        pg = u[3 * H:] * (one + u[H:2 * H])
        c2_new = c[l] * tfh.astype(f32) + pg.astype(f32)
        tc = jnp.tanh(c2_new.astype(bf16) * half)
        h2 = u[2 * H:3 * H] * tc + tc
        c[l] = c2_new
        h_bf[l] = h2
        if l < L - 1:
            below[l][t] = h2
        else:
            h_top[0] = h2.astype(f32)

    for d in range(L + T - 1):
        for l in range(L):
            t = d - l
            if 0 <= t < T:
                cell(l, t)

    h_last = h_top[0]
    mu = jnp.mean(h_last, axis=0, keepdims=True)
    var = jnp.mean((h_last - mu) ** 2, axis=0, keepdims=True)
    xn = (h_last - mu) * jax.lax.rsqrt(var + LN_EPS * 4.0)
    xn = xn * ln_g_ref[...] + ln_b_ref[...]

    out = jnp.dot(w_out_ref[...], xn, preferred_element_type=f32) + b_out_ref[...]
    out_ref[...] = out.astype(out_ref.dtype)


@jax.jit
def kernel(x, w_ih0, w_hh0, w_fused, b, ln_g, ln_b, w_out, b_out):
    B, T, F = x.shape
    H, L = HIDDEN, NUM_LAYERS
    G = 4 * H
    bt = BT if B >= BT else max(8, (B + 7) // 8 * 8)
    nb = pl.cdiv(B, bt)
    b_pad = bt * nb

    x_p = jnp.zeros((T, F_PAD, b_pad), jnp.bfloat16)
    x_p = x_p.at[:, :F, :B].set(jnp.transpose(x.astype(jnp.bfloat16), (1, 2, 0)))
    x_p = x_p.at[:, F, :].set(jnp.bfloat16(1.0))

    gate_scale = jnp.concatenate(
        [jnp.full((3 * H, 1), 0.5, jnp.float32),
         jnp.ones((H, 1), jnp.float32)], axis=0)
    h_in_scale = jnp.concatenate(
        [jnp.full((1, H), 0.5, jnp.float32),
         jnp.ones((1, F_PAD), jnp.float32)], axis=1)
    w0_rows = jnp.concatenate(
        [w_hh0.astype(jnp.float32),
         w_ih0[:F].astype(jnp.float32),
         b[0],
         jnp.zeros((1, G), jnp.float32)], axis=0)
    w0t = (w0_rows.T * gate_scale * h_in_scale).astype(jnp.bfloat16)
    wft = (jnp.transpose(w_fused, (0, 2, 1))
           * (gate_scale * 0.5).astype(jnp.bfloat16))
    b_t = (jnp.transpose(b, (0, 2, 1)) * gate_scale).astype(jnp.bfloat16)

    flops = (2 * b_pad * T * (F_PAD + H) * G
             + 2 * b_pad * T * (L - 1) * 2 * H * G
             + 2 * b_pad * H * LABEL)
    transcendentals = b_pad * T * L * 5 * H
    bytes_accessed = int(x_p.size * 2 + w_fused.size * 2 + b_pad * LABEL * 4)

    def resident(a):
        nd = a.ndim
        return pl.BlockSpec(a.shape, lambda i, nd=nd: (0,) * nd)

    out = pl.pallas_call(
        _lstm_wavefront_kernel,
        out_shape=jax.ShapeDtypeStruct((LABEL, b_pad), jnp.float32),
        grid=(nb,),
        in_specs=[
            pl.BlockSpec((T, F_PAD, bt), lambda i: (0, 0, i)),
            resident(w0t),
            resident(wft),
            resident(b_t),
            resident(ln_g.T),
            resident(ln_b.T),
            resident(w_out.T),
            resident(b_out.T),
        ],
        out_specs=pl.BlockSpec((LABEL, bt), lambda i: (0, i)),
        cost_estimate=pl.CostEstimate(flops=flops,
                                      transcendentals=transcendentals,
                                      bytes_accessed=bytes_accessed),
        compiler_params=pltpu.CompilerParams(
            dimension_semantics=("parallel",)),
    )(x_p, w0t, wft, b_t, ln_g.T, ln_b.T, w_out.T, b_out.T)
    return out[:, :B].T


# speedup vs baseline: 6.2998x; 1.0862x over previous
import jax
import jax.numpy as jnp
from jax.experimental import pallas as pl
from jax.experimental.pallas import tpu as pltpu

IN_FEATS = 6
F_PAD = 8
HIDDEN = 64
NUM_LAYERS = 4
LABEL = 20
LN_EPS = 1e-5
BT = 1024


def _lstm_wavefront_kernel(x_ref, w0t_ref, wft_ref, bt_ref, ln_g_ref,
                           ln_b_ref, w_out_ref, b_out_ref, out_ref):
    Bt = x_ref.shape[0]
    T = x_ref.shape[1] // IN_FEATS
    L, H = NUM_LAYERS, HIDDEN
    f32 = jnp.float32
    bf16 = jnp.bfloat16

    w0t = w0t_ref[...]
    wft = [wft_ref[l] for l in range(L - 1)]
    bias = [bt_ref[l] for l in range(L)]

    h_bf = [jnp.zeros((H, Bt), bf16) for _ in range(L)]
    c = [jnp.zeros((H, Bt), f32) for _ in range(L)]
    below = [[None] * T for _ in range(L - 1)]
    h_top = [jnp.zeros((H, Bt), f32)]

    one = jnp.bfloat16(1.0)
    half = jnp.bfloat16(0.5)

    xt = x_ref[...].T

    def cell(l, t):
        if l == 0:
            hin = jnp.concatenate(
                [h_bf[0], xt[IN_FEATS * t:IN_FEATS * (t + 1)]], axis=0)
            gb = (jnp.dot(w0t, hin, preferred_element_type=f32)
                  .astype(bf16) + bias[0])
        else:
            hin = jnp.concatenate([below[l - 1][t], h_bf[l]], axis=0)
            gb = (jnp.dot(wft[l - 1], hin, preferred_element_type=f32)
                  .astype(bf16) + bias[l])

        u = jnp.tanh(gb)
        tfh = (one + u[:H]) * half
        pg = u[3 * H:] * (one + u[H:2 * H])
        c2_new = c[l] * tfh.astype(f32) + pg.astype(f32)
        tc = jnp.tanh(c2_new.astype(bf16) * half)
        h2 = u[2 * H:3 * H] * tc + tc
        c[l] = c2_new
        h_bf[l] = h2
        if l < L - 1:
            below[l][t] = h2
        else:
            h_top[0] = h2.astype(f32)

    for d in range(L + T - 1):
        for l in range(L):
            t = d - l
            if 0 <= t < T:
                cell(l, t)

    h_last = h_top[0]
    mu = jnp.mean(h_last, axis=0, keepdims=True)
    var = jnp.mean((h_last - mu) ** 2, axis=0, keepdims=True)
    xn = (h_last - mu) * jax.lax.rsqrt(var + LN_EPS * 4.0)
    xn = xn * ln_g_ref[...] + ln_b_ref[...]

    out = jnp.dot(w_out_ref[...], xn, preferred_element_type=f32) + b_out_ref[...]
    out_ref[...] = out.astype(out_ref.dtype)


@jax.jit
def kernel(x, w_ih0, w_hh0, w_fused, b, ln_g, ln_b, w_out, b_out):
    B, T, F = x.shape
    H, L = HIDDEN, NUM_LAYERS
    G = 4 * H
    bt = BT if B >= BT else max(8, (B + 7) // 8 * 8)
    nb = pl.cdiv(B, bt)
    b_pad = bt * nb

    x_flat = x.reshape(B, T * F).astype(jnp.bfloat16)
    if b_pad != B:
        x_flat = jnp.zeros((b_pad, T * F), jnp.bfloat16).at[:B].set(x_flat)

    gate_scale = jnp.concatenate(
        [jnp.full((3 * H, 1), 0.5, jnp.float32),
         jnp.ones((H, 1), jnp.float32)], axis=0)
    h_in_scale = jnp.concatenate(
        [jnp.full((1, H), 0.5, jnp.float32),
         jnp.ones((1, F), jnp.float32)], axis=1)
    w0_rows = jnp.concatenate(
        [w_hh0.astype(jnp.float32),
         w_ih0[:F].astype(jnp.float32)], axis=0)
    w0t = (w0_rows.T * gate_scale * h_in_scale).astype(jnp.bfloat16)
    wft = (jnp.transpose(w_fused, (0, 2, 1))
           * (gate_scale * 0.5).astype(jnp.bfloat16))
    b_t = (jnp.transpose(b, (0, 2, 1)) * gate_scale).astype(jnp.bfloat16)

    flops = (2 * b_pad * T * (F_PAD + H) * G
             + 2 * b_pad * T * (L - 1) * 2 * H * G
             + 2 * b_pad * H * LABEL)
    transcendentals = b_pad * T * L * 5 * H
    bytes_accessed = int(x_flat.size * 2 + w_fused.size * 2 + b_pad * LABEL * 4)

    def resident(a):
        nd = a.ndim
        return pl.BlockSpec(a.shape, lambda i, nd=nd: (0,) * nd)

    out = pl.pallas_call(
        _lstm_wavefront_kernel,
        out_shape=jax.ShapeDtypeStruct((LABEL, b_pad), jnp.float32),
        grid=(nb,),
        in_specs=[
            pl.BlockSpec((bt, T * F), lambda i: (i, 0)),
            resident(w0t),
            resident(wft),
            resident(b_t),
            resident(ln_g.T),
            resident(ln_b.T),
            resident(w_out.T),
            resident(b_out.T),
        ],
        out_specs=pl.BlockSpec((LABEL, bt), lambda i: (0, i)),
        cost_estimate=pl.CostEstimate(flops=flops,
                                      transcendentals=transcendentals,
                                      bytes_accessed=bytes_accessed),
        compiler_params=pltpu.CompilerParams(
            dimension_semantics=("parallel",)),
    )(x_flat, w0t, wft, b_t, ln_g.T, ln_b.T, w_out.T, b_out.T)
    return out[:, :B].T
